# baseline (speedup 1.0000x reference)
import jax
import jax.numpy as jnp
from jax import lax
from jax.experimental import pallas as pl

R = 16
INT = 8
WREP = 49.0
HW = 64
NPIX = HW * HW
C = 128
HID = 512
NB = 4
NT = 3
BATCH = 2
ROWS = BATCH * R * R
KEYS = R * R
NPIXB = BATCH * NPIX
NORM = float(BATCH * NPIX)


def _shift_rows(x, off):
    if off == 0:
        return x
    z = jnp.zeros((abs(off), x.shape[1]), x.dtype)
    if off > 0:
        return jnp.concatenate([x[off:], z], axis=0)
    return jnp.concatenate([z, x[:off]], axis=0)


def _shift4(x, d, axis):
    if d == 0:
        return x
    n = x.shape[axis]
    zshape = list(x.shape)
    zshape[axis] = abs(d)
    z = jnp.zeros(zshape, x.dtype)
    if d > 0:
        return jnp.concatenate([lax.slice_in_dim(x, d, n, axis=axis), z],
                               axis=axis)
    return jnp.concatenate([z, lax.slice_in_dim(x, 0, n + d, axis=axis)],
                           axis=axis)


def _towers_body(temb_ref, fcw_ref, fcb_ref, w1_ref, b1_ref, dw_ref,
                 dwb_ref, w2_ref, b2_ref, bng_ref, bnb_ref):
    s = lax.broadcasted_iota(jnp.int32, (ROWS, 1), 0)
    b_id = s // (R * R)
    sl = lax.broadcasted_iota(jnp.int32, (1, ROWS), 1)
    hl = (sl // R) % R
    wl = sl % R
    wt_l = (jnp.where(hl == INT, WREP, 1.0)
            * jnp.where(wl == INT, WREP, 1.0))
    hiprec = jax.lax.Precision.HIGHEST

    prod = None
    for t in range(NT):
        e = jnp.mean(temb_ref[t], axis=1)
        x0 = jax.nn.relu(
            lax.dot_general(e, fcw_ref[t], (((1,), (1,)), ((), ())),
                            preferred_element_type=jnp.float32)
            + fcb_ref[t:t + 1])
        x = jnp.where(b_id == 0, x0[0:1], x0[1:2])

        for k in range(NB):
            mu = lax.dot_general(wt_l, x, (((1,), (0,)), ((), ())),
                                 preferred_element_type=jnp.float32,
                                 precision=hiprec) / NORM
            ex2 = lax.dot_general(wt_l, x * x, (((1,), (0,)), ((), ())),
                                  preferred_element_type=jnp.float32,
                                  precision=hiprec) / NORM
            var = ex2 - mu * mu
            xn = (x - mu) / jnp.sqrt(var + 1e-5)
            xn = xn * bng_ref[t, k:k + 1] + bnb_ref[t, k:k + 1]
            h = jax.nn.relu(
                lax.dot_general(xn, w1_ref[t, k], (((1,), (1,)), ((), ())),
                                preferred_element_type=jnp.float32)
                + b1_ref[t, k:k + 1])
            h4 = h.reshape(BATCH, R, R, HID)
            h4 = jnp.concatenate([h4[:, :, R - 1:, :], h4[:, :, :R - 1, :]],
                                 axis=2)
            acc = None
            for ky in range(3):
                hy = _shift4(h4, ky - 1, 1)
                for kx in range(3):
                    kv = dw_ref[t, k, 3 * ky + kx:3 * ky + kx + 1]
                    term = _shift4(hy, kx - 1, 2) * kv
                    acc = term if acc is None else acc + term
            h = jax.nn.relu(acc + dwb_ref[t, k:k + 1]).reshape(ROWS, HID)
            x = (x
                 + lax.dot_general(h, w2_ref[t, k], (((1,), (1,)), ((), ())),
                                   preferred_element_type=jnp.float32)
                 + b2_ref[t, k:k + 1])
        prod = x if prod is None else prod * x

    return prod / (jnp.sqrt(jnp.sum(prod * prod, axis=1,
                                    keepdims=True)) + 1e-6)


def _fused_kernel(v_ref, temb_ref, fcw_ref, fcb_ref, w1_ref, b1_ref, dw_ref,
                  dwb_ref, w2_ref, b2_ref, bng_ref, bnb_ref,
                  tvw1_ref, tvb1_ref, tvw2_ref, tvb2_ref,
                  ttw1_ref, ttb1_ref, ttw2_ref, ttb2_ref, o_ref):
    keys = _towers_body(temb_ref, fcw_ref, fcb_ref, w1_ref, b1_ref, dw_ref,
                        dwb_ref, w2_ref, b2_ref, bng_ref, bnb_ref)

    vf = v_ref[...]
    pn = vf / (jnp.sqrt(jnp.sum(vf * vf, axis=1, keepdims=True)) + 1e-6)
    pn_aug = jnp.concatenate(
        [pn * -2.0, jnp.ones((NPIXB, 1), jnp.float32)], axis=1)
    kn2 = jnp.sum(keys * keys, axis=1, keepdims=True)
    keys_aug = jnp.concatenate([keys, kn2], axis=1)
    trs = []
    for b in range(BATCH):
        ka = keys_aug[b * KEYS:(b + 1) * KEYS]
        d2 = lax.dot_general(pn_aug[b * NPIX:(b + 1) * NPIX], ka,
                             (((1,), (1,)), ((), ())),
                             preferred_element_type=jnp.float32)
        m = jnp.min(d2, axis=1, keepdims=True)
        ji = lax.broadcasted_iota(jnp.int32, d2.shape, 1)
        idx = jnp.min(jnp.where(d2 == m, ji, KEYS), axis=1, keepdims=True)
        onehot = (ji == idx).astype(jnp.float32)
        trs.append(lax.dot_general(onehot, keys[b * KEYS:(b + 1) * KEYS],
                                   (((1,), (0,)), ((), ())),
                                   preferred_element_type=jnp.float32))
    tf = jnp.concatenate(trs, axis=0)

    s = lax.broadcasted_iota(jnp.int32, (NPIXB, 1), 0)
    hpos = (s // HW) % HW
    wpos = s % HW

    def blend_coef(vcur, stride, pos):
        tprev = _shift_rows(tf, -stride)
        num = jnp.sum(vcur * tprev, axis=1, keepdims=True)
        den = jnp.maximum(
            jnp.sqrt(jnp.sum(vcur * vcur, axis=1, keepdims=True))
            * jnp.sqrt(jnp.sum(tprev * tprev, axis=1, keepdims=True)), 1e-8)
        return jnp.where(pos == 0, 0.0, jnp.exp(-(1.0 - num / den)))

    def linscan(vcur, stride, pos):
        A = blend_coef(vcur, stride, pos)
        Bv = (1.0 - A) * vcur
        k = 1
        while k < HW:
            live = pos >= k
            Am = jnp.where(live, A, 0.0)
            Bv = Am * _shift_rows(Bv, -k * stride) + Bv
            A = A * jnp.where(live, _shift_rows(A, -k * stride), 1.0)
            k *= 2
        return Bv

    vr = linscan(vf, 1, wpos)
    vc = linscan(vr, HW, hpos)

    def cosd(a, b):
        num = jnp.sum(a * b, axis=1, keepdims=True)
        den = jnp.maximum(
            jnp.sqrt(jnp.sum(a * a, axis=1, keepdims=True))
            * jnp.sqrt(jnp.sum(b * b, axis=1, keepdims=True)), 1e-8)
        return 1.0 - num / den

    d_tv = cosd(vc, tf)
    tnext = _shift_rows(tf, 1)
    d_tt = jnp.where(s % NPIX == NPIX - 1, 0.0, cosd(tf, tnext))

    def mlp(d, w1, b1, w2, b2):
        h = jax.nn.relu(d * w1 + b1)
        return jnp.sum(h * w2, axis=1, keepdims=True) + b2

    gate = jax.nn.sigmoid(
        mlp(d_tv, tvw1_ref[...], tvb1_ref[...], tvw2_ref[...], tvb2_ref[...])
        + mlp(d_tt, ttw1_ref[...], ttb1_ref[...], ttw2_ref[...], ttb2_ref[...]))
    o_ref[...] = vc * gate


def kernel(V, tA, tB, tAB, params):
    towers = [params[n] for n in ('tA', 'tB', 'tAB')]
    temb = jnp.stack([tA, tB, tAB])
    fcw = jnp.stack([p['fc_w'] for p in towers])
    fcb = jnp.stack([p['fc_b'] for p in towers])

    def blk(name):
        return jnp.stack([jnp.stack([b[name] for b in p['blocks']])
                          for p in towers])

    w1, b1, dwb = blk('w1'), blk('b1'), blk('dwb')
    w2, b2 = blk('w2'), blk('b2')
    bng, bnb = blk('bn_g'), blk('bn_b')
    dw = blk('dw').reshape(NT, NB, HID, 9).transpose(0, 1, 3, 2)

    vn = jnp.transpose(V, (0, 2, 3, 1)).reshape(NPIXB, C)
    mlp_params = (
        params['tv']['w1'].reshape(1, 256), params['tv']['b1'].reshape(1, 256),
        params['tv']['w2'].reshape(1, 256), params['tv']['b2'].reshape(1, 1),
        params['tt']['w1'].reshape(1, 256), params['tt']['b1'].reshape(1, 256),
        params['tt']['w2'].reshape(1, 256), params['tt']['b2'].reshape(1, 1),
    )
    out = pl.pallas_call(
        _fused_kernel,
        out_shape=jax.ShapeDtypeStruct((NPIXB, C), jnp.float32),
    )(vn, temb, fcw, fcb, w1, b1, dw, dwb, w2, b2, bng, bnb, *mlp_params)
    return jnp.transpose(out.reshape(BATCH, HW, HW, C), (0, 3, 1, 2))
